# baseline (speedup 1.0000x reference)
import functools

import jax
import jax.numpy as jnp
from jax import lax
from jax.experimental import pallas as pl
from jax.experimental.pallas import tpu as pltpu
from jax.experimental.pallas import tpu_sc as plsc

N = 10000
D = 256
E = 160000
NP = 10240
EPAD = 163840
TRASH = 10000
HALF = 128
NH = NP // 2
QROWS = 5248
QSTRIPE = QROWS // 16
OSTRIPE = NH // 16
STRIPE = NP // 16
BN = 512
_NCHUNK = 80
_NB = 4


def _deg_body(dst32, zeros1, deg_out, idx_v, ones_v, deg_sh):
    c = lax.axis_index("c")
    s = lax.axis_index("s")
    w = c * 16 + s
    pltpu.sync_copy(zeros1.at[pl.ds(s * STRIPE, STRIPE)],
                    deg_sh.at[pl.ds(s * STRIPE, STRIPE)])
    pltpu.sync_copy(dst32.at[w], idx_v)
    for k in range(8):
        ones_v[pl.ds(k * 16, 16)] = jnp.full((16,), 1.0, jnp.float32)
    plsc.subcore_barrier()

    def body(j, carry):
        pltpu.sync_copy(ones_v, deg_sh.at[idx_v.at[j]], add=True)
        return carry

    lax.fori_loop(0, 40, body, 0)
    plsc.subcore_barrier()
    pltpu.sync_copy(deg_sh.at[pl.ds(s * STRIPE, STRIPE)],
                    deg_out.at[c, pl.ds(s * STRIPE, STRIPE)])


def _agg_body(g_hbm, src2, dstq, zerosq, acc_out, idxs_v, idxd_v,
              buf0, buf1, buf2, buf3, acc_sh, sg, ss):
    c = lax.axis_index("c")
    s = lax.axis_index("s")
    bufs = (buf0, buf1, buf2, buf3)
    pltpu.sync_copy(src2.at[c, s], idxs_v)
    for qn in range(2):
        pltpu.sync_copy(zerosq.at[pl.ds(s * QSTRIPE, QSTRIPE)],
                        acc_sh.at[pl.ds(s * QSTRIPE, QSTRIPE)])
        pltpu.sync_copy(dstq.at[qn, s], idxd_v)
        plsc.subcore_barrier()

        def body(i, carry):
            j = i * _NB
            for b, buf in enumerate(bufs):
                pltpu.async_copy(g_hbm.at[idxs_v.at[j + b]], buf, sg)
            for buf in bufs:
                pltpu.make_async_copy(g_hbm.at[idxs_v.at[0]], buf, sg).wait()
            for b, buf in enumerate(bufs):
                pltpu.async_copy(buf, acc_sh.at[idxd_v.at[j + b]], ss,
                                 add=True)
            for buf in bufs:
                pltpu.make_async_copy(g_hbm.at[idxs_v.at[0]], buf, ss).wait()
            return carry

        lax.fori_loop(0, _NCHUNK // _NB, body, 0)
        plsc.subcore_barrier()
        pltpu.sync_copy(acc_sh.at[pl.ds(s * OSTRIPE, OSTRIPE)],
                        acc_out.at[c, pl.ds(qn * NH + s * OSTRIPE, OSTRIPE)])
        plsc.subcore_barrier()


@functools.cache
def _sc_kernels():
    mesh = plsc.VectorSubcoreMesh(core_axis_name="c", subcore_axis_name="s",
                                  num_cores=2, num_subcores=16)
    deg = pl.kernel(
        _deg_body,
        out_type=jax.ShapeDtypeStruct((2, NP), jnp.float32),
        mesh=mesh,
        scratch_types=[
            pltpu.VMEM((40, 128), jnp.int32),
            pltpu.VMEM((128,), jnp.float32),
            pltpu.VMEM_SHARED((NP,), jnp.float32),
        ],
    )
    agg = pl.kernel(
        _agg_body,
        out_type=jax.ShapeDtypeStruct((2, NP, HALF), jnp.float32),
        mesh=mesh,
        scratch_types=[
            pltpu.VMEM((_NCHUNK, 128), jnp.int32),
            pltpu.VMEM((_NCHUNK, 128), jnp.int32),
            pltpu.VMEM((128, HALF), jnp.float32),
            pltpu.VMEM((128, HALF), jnp.float32),
            pltpu.VMEM((128, HALF), jnp.float32),
            pltpu.VMEM((128, HALF), jnp.float32),
            pltpu.VMEM_SHARED((QROWS, HALF), jnp.float32),
            pltpu.SemaphoreType.DMA,
            pltpu.SemaphoreType.DMA,
        ],
    )
    return deg, agg


def _dinv(d0_ref, d1_ref):
    return lax.rsqrt(d0_ref[...] + d1_ref[...] + 1.0)


def _leaky(z):
    return jnp.where(z >= 0, z, 0.01 * z)


def _mm_body(x_ref, w_ref, d0_ref, d1_ref, g_ref):
    u = x_ref[...] * _dinv(d0_ref, d1_ref)
    h = jnp.dot(u, w_ref[...], preferred_element_type=jnp.float32)
    g_ref[0] = h[:, :HALF]
    g_ref[1] = h[:, HALF:]


def _mid_body(acc_ref, g_ref, d0_ref, d1_ref, b_ref, w_ref, out_ref):
    dinv = _dinv(d0_ref, d1_ref)
    z0 = _leaky(dinv * (acc_ref[0] + g_ref[0]) + b_ref[0:1, :HALF])
    z1 = _leaky(dinv * (acc_ref[1] + g_ref[1]) + b_ref[0:1, HALF:])
    h = (jnp.dot(dinv * z0, w_ref[:HALF, :], preferred_element_type=jnp.float32)
         + jnp.dot(dinv * z1, w_ref[HALF:, :], preferred_element_type=jnp.float32))
    out_ref[0] = h[:, :HALF]
    out_ref[1] = h[:, HALF:]


def _fin_body(acc_ref, g_ref, d0_ref, d1_ref, b_ref, out_ref):
    dinv = _dinv(d0_ref, d1_ref)
    out_ref[:, :HALF] = _leaky(dinv * (acc_ref[0] + g_ref[0]) + b_ref[0:1, :HALF])
    out_ref[:, HALF:] = _leaky(dinv * (acc_ref[1] + g_ref[1]) + b_ref[0:1, HALF:])


_GRID = (NP // BN,)
_SPEC_ROWS = pl.BlockSpec((BN, D), lambda i: (i, 0))
_SPEC_W = pl.BlockSpec((D, D), lambda i: (0, 0))
_SPEC_D = pl.BlockSpec((BN, 1), lambda i: (i, 0))
_SPEC_B = pl.BlockSpec((1, D), lambda i: (0, 0))
_SPEC_HALVES = pl.BlockSpec((2, BN, HALF), lambda i: (0, i, 0))

_mm_call = pl.pallas_call(
    _mm_body,
    grid=_GRID,
    in_specs=[_SPEC_ROWS, _SPEC_W, _SPEC_D, _SPEC_D],
    out_specs=_SPEC_HALVES,
    out_shape=jax.ShapeDtypeStruct((2, NP, HALF), jnp.float32),
)

_mid_call = pl.pallas_call(
    _mid_body,
    grid=_GRID,
    in_specs=[_SPEC_HALVES, _SPEC_HALVES, _SPEC_D, _SPEC_D, _SPEC_B,
              _SPEC_W],
    out_specs=_SPEC_HALVES,
    out_shape=jax.ShapeDtypeStruct((2, NP, HALF), jnp.float32),
)

_fin_call = pl.pallas_call(
    _fin_body,
    grid=_GRID,
    in_specs=[_SPEC_HALVES, _SPEC_HALVES, _SPEC_D, _SPEC_D, _SPEC_B],
    out_specs=_SPEC_ROWS,
    out_shape=jax.ShapeDtypeStruct((NP, D), jnp.float32),
)


def kernel(x, edge_index, W1, b1, W2, b2):
    ei = edge_index.astype(jnp.int32)
    srcp = jnp.concatenate([ei[0], jnp.zeros((EPAD - E,), jnp.int32)])
    dstp = jnp.concatenate([ei[1], jnp.full((EPAD - E,), TRASH, jnp.int32)])
    src2 = jnp.stack([srcp, srcp + NP]).reshape(2, 16, _NCHUNK, 128)
    dq = []
    for qn in range(2):
        rel = dstp - qn * NH
        dq.append(jnp.where((rel >= 0) & (rel < NH), rel, NH))
    dstq = jnp.stack(dq).reshape(2, 16, _NCHUNK, 128)
    dst32 = dstp.reshape(32, 40, 128)
    xpad = jnp.zeros((NP, D), jnp.float32).at[:N].set(x)
    zeros1 = jnp.zeros((NP,), jnp.float32)
    zerosq = jnp.zeros((QROWS, HALF), jnp.float32)
    b1r = b1.reshape(1, D)
    b2r = b2.reshape(1, D)

    _deg, _agg = _sc_kernels()
    deg = _deg(dst32, zeros1)
    d0 = deg[0][:, None]
    d1 = deg[1][:, None]
    g1 = _mm_call(xpad, W1, d0, d1)
    acc1 = _agg(g1.reshape(2 * NP, HALF), src2, dstq, zerosq)
    g2 = _mid_call(acc1, g1, d0, d1, b1r, W2)
    acc2 = _agg(g2.reshape(2 * NP, HALF), src2, dstq, zerosq)
    out = _fin_call(acc2, g2, d0, d1, b2r)
    return out[:N]


# speedup vs baseline: 1.9041x; 1.9041x over previous
import functools

import jax
import jax.numpy as jnp
from jax import lax
from jax.experimental import pallas as pl
from jax.experimental.pallas import tpu as pltpu
from jax.experimental.pallas import tpu_sc as plsc

N = 10000
D = 256
E = 160000
NP = 10240
EPAD = 163840
TRASH = 10000
HALF = 128
NH = NP // 2
QROWS = 5248
QSTRIPE = QROWS // 16
OSTRIPE = NH // 16
STRIPE = NP // 16
BN = 512
_NCHUNK = 80
_NB = 4


def _deg_body(dst32, zeros1, deg_out, idx_v, ones_v, deg_sh):
    c = lax.axis_index("c")
    s = lax.axis_index("s")
    w = c * 16 + s
    pltpu.sync_copy(zeros1.at[pl.ds(s * STRIPE, STRIPE)],
                    deg_sh.at[pl.ds(s * STRIPE, STRIPE)])
    pltpu.sync_copy(dst32.at[w], idx_v)
    for k in range(8):
        ones_v[pl.ds(k * 16, 16)] = jnp.full((16,), 1.0, jnp.float32)
    plsc.subcore_barrier()

    def body(j, carry):
        pltpu.sync_copy(ones_v, deg_sh.at[idx_v.at[j]], add=True)
        return carry

    lax.fori_loop(0, 40, body, 0)
    plsc.subcore_barrier()
    pltpu.sync_copy(deg_sh.at[pl.ds(s * STRIPE, STRIPE)],
                    deg_out.at[c, pl.ds(s * STRIPE, STRIPE)])


def _agg_body(g_hbm, src2, dst16, zeros2, acc_out, idxs_v, idxd_v,
              rows_v, acc_sh, sem):
    c = lax.axis_index("c")
    s = lax.axis_index("s")
    pltpu.sync_copy(zeros2.at[pl.ds(s * STRIPE, STRIPE)],
                    acc_sh.at[pl.ds(s * STRIPE, STRIPE)])
    pltpu.sync_copy(src2.at[c, s], idxs_v)
    pltpu.sync_copy(dst16.at[s], idxd_v)
    plsc.subcore_barrier()

    def body(j, carry):
        pltpu.async_copy(g_hbm.at[idxs_v.at[j]], rows_v, sem).wait()
        pltpu.sync_copy(rows_v, acc_sh.at[idxd_v.at[j]], add=True)
        return carry

    lax.fori_loop(0, _NCHUNK, body, 0)
    plsc.subcore_barrier()
    pltpu.sync_copy(acc_sh.at[pl.ds(s * STRIPE, STRIPE)],
                    acc_out.at[c, pl.ds(s * STRIPE, STRIPE)])


@functools.cache
def _sc_kernels():
    mesh = plsc.VectorSubcoreMesh(core_axis_name="c", subcore_axis_name="s",
                                  num_cores=2, num_subcores=16)
    deg = pl.kernel(
        _deg_body,
        out_type=jax.ShapeDtypeStruct((2, NP), jnp.float32),
        mesh=mesh,
        scratch_types=[
            pltpu.VMEM((40, 128), jnp.int32),
            pltpu.VMEM((128,), jnp.float32),
            pltpu.VMEM_SHARED((NP,), jnp.float32),
        ],
    )
    agg = pl.kernel(
        _agg_body,
        out_type=jax.ShapeDtypeStruct((2, NP, HALF), jnp.float32),
        mesh=mesh,
        scratch_types=[
            pltpu.VMEM((_NCHUNK, 128), jnp.int32),
            pltpu.VMEM((_NCHUNK, 128), jnp.int32),
            pltpu.VMEM((128, HALF), jnp.float32),
            pltpu.VMEM_SHARED((NP, HALF), jnp.float32),
            pltpu.SemaphoreType.DMA,
        ],
    )
    return deg, agg


def _dinv(d0_ref, d1_ref):
    return lax.rsqrt(d0_ref[...] + d1_ref[...] + 1.0)


def _leaky(z):
    return jnp.where(z >= 0, z, 0.01 * z)


def _mm_body(x_ref, w_ref, d0_ref, d1_ref, g_ref):
    u = x_ref[...] * _dinv(d0_ref, d1_ref)
    h = jnp.dot(u, w_ref[...], preferred_element_type=jnp.float32)
    g_ref[0] = h[:, :HALF]
    g_ref[1] = h[:, HALF:]


def _mid_body(acc_ref, g_ref, d0_ref, d1_ref, b_ref, w_ref, out_ref):
    dinv = _dinv(d0_ref, d1_ref)
    z0 = _leaky(dinv * (acc_ref[0] + g_ref[0]) + b_ref[0:1, :HALF])
    z1 = _leaky(dinv * (acc_ref[1] + g_ref[1]) + b_ref[0:1, HALF:])
    h = (jnp.dot(dinv * z0, w_ref[:HALF, :], preferred_element_type=jnp.float32)
         + jnp.dot(dinv * z1, w_ref[HALF:, :], preferred_element_type=jnp.float32))
    out_ref[0] = h[:, :HALF]
    out_ref[1] = h[:, HALF:]


def _fin_body(acc_ref, g_ref, d0_ref, d1_ref, b_ref, out_ref):
    dinv = _dinv(d0_ref, d1_ref)
    out_ref[:, :HALF] = _leaky(dinv * (acc_ref[0] + g_ref[0]) + b_ref[0:1, :HALF])
    out_ref[:, HALF:] = _leaky(dinv * (acc_ref[1] + g_ref[1]) + b_ref[0:1, HALF:])


_GRID = (NP // BN,)
_SPEC_ROWS = pl.BlockSpec((BN, D), lambda i: (i, 0))
_SPEC_W = pl.BlockSpec((D, D), lambda i: (0, 0))
_SPEC_D = pl.BlockSpec((BN, 1), lambda i: (i, 0))
_SPEC_B = pl.BlockSpec((1, D), lambda i: (0, 0))
_SPEC_HALVES = pl.BlockSpec((2, BN, HALF), lambda i: (0, i, 0))

_mm_call = pl.pallas_call(
    _mm_body,
    grid=_GRID,
    in_specs=[_SPEC_ROWS, _SPEC_W, _SPEC_D, _SPEC_D],
    out_specs=_SPEC_HALVES,
    out_shape=jax.ShapeDtypeStruct((2, NP, HALF), jnp.float32),
)

_mid_call = pl.pallas_call(
    _mid_body,
    grid=_GRID,
    in_specs=[_SPEC_HALVES, _SPEC_HALVES, _SPEC_D, _SPEC_D, _SPEC_B,
              _SPEC_W],
    out_specs=_SPEC_HALVES,
    out_shape=jax.ShapeDtypeStruct((2, NP, HALF), jnp.float32),
)

_fin_call = pl.pallas_call(
    _fin_body,
    grid=_GRID,
    in_specs=[_SPEC_HALVES, _SPEC_HALVES, _SPEC_D, _SPEC_D, _SPEC_B],
    out_specs=_SPEC_ROWS,
    out_shape=jax.ShapeDtypeStruct((NP, D), jnp.float32),
)


def kernel(x, edge_index, W1, b1, W2, b2):
    ei = edge_index.astype(jnp.int32)
    srcp = jnp.concatenate([ei[0], jnp.zeros((EPAD - E,), jnp.int32)])
    dstp = jnp.concatenate([ei[1], jnp.full((EPAD - E,), TRASH, jnp.int32)])
    src2 = jnp.stack([srcp, srcp + NP]).reshape(2, 16, _NCHUNK, 128)
    dst16 = dstp.reshape(16, _NCHUNK, 128)
    dst32 = dstp.reshape(32, 40, 128)
    xpad = jnp.zeros((NP, D), jnp.float32).at[:N].set(x)
    zeros1 = jnp.zeros((NP,), jnp.float32)
    zeros2 = jnp.zeros((NP, HALF), jnp.float32)
    b1r = b1.reshape(1, D)
    b2r = b2.reshape(1, D)

    _deg, _agg = _sc_kernels()
    deg = _deg(dst32, zeros1)
    d0 = deg[0][:, None]
    d1 = deg[1][:, None]
    g1 = _mm_call(xpad, W1, d0, d1)
    acc1 = _agg(g1.reshape(2 * NP, HALF), src2, dst16, zeros2)
    g2 = _mid_call(acc1, g1, d0, d1, b1r, W2)
    acc2 = _agg(g2.reshape(2 * NP, HALF), src2, dst16, zeros2)
    out = _fin_call(acc2, g2, d0, d1, b2r)
    return out[:N]


# speedup vs baseline: 1.9057x; 1.0008x over previous
import functools

import jax
import jax.numpy as jnp
from jax import lax
from jax.experimental import pallas as pl
from jax.experimental.pallas import tpu as pltpu
from jax.experimental.pallas import tpu_sc as plsc

N = 10000
D = 256
E = 160000
NP = 10240
EPAD = 163840
TRASH = 10000
HALF = 128
STRIPE = NP // 16
BN = 512
_NCHUNK = 80


def _deg_body(dst32, zeros1, deg_out, idx_v, ones_v, deg_sh):
    c = lax.axis_index("c")
    s = lax.axis_index("s")
    w = c * 16 + s
    pltpu.sync_copy(zeros1.at[pl.ds(s * STRIPE, STRIPE)],
                    deg_sh.at[pl.ds(s * STRIPE, STRIPE)])
    pltpu.sync_copy(dst32.at[w], idx_v)
    for k in range(8):
        ones_v[pl.ds(k * 16, 16)] = jnp.full((16,), 1.0, jnp.float32)
    plsc.subcore_barrier()

    def body(j, carry):
        pltpu.sync_copy(ones_v, deg_sh.at[idx_v.at[j]], add=True)
        return carry

    lax.fori_loop(0, 40, body, 0)
    plsc.subcore_barrier()
    pltpu.sync_copy(deg_sh.at[pl.ds(s * STRIPE, STRIPE)],
                    deg_out.at[c, pl.ds(s * STRIPE, STRIPE)])


def _agg_body(g_hbm, src2, dst16, zeros2, acc_out, idxs_v, idxd_v,
              rows_v, acc_sh, sem):
    c = lax.axis_index("c")
    s = lax.axis_index("s")
    pltpu.sync_copy(zeros2.at[pl.ds(s * STRIPE, STRIPE)],
                    acc_sh.at[pl.ds(s * STRIPE, STRIPE)])
    pltpu.sync_copy(src2.at[c, s], idxs_v)
    pltpu.sync_copy(dst16.at[s], idxd_v)
    plsc.subcore_barrier()

    def body(j, carry):
        pltpu.async_copy(g_hbm.at[idxs_v.at[j]], rows_v, sem).wait()
        pltpu.sync_copy(rows_v, acc_sh.at[idxd_v.at[j]], add=True)
        return carry

    lax.fori_loop(0, _NCHUNK, body, 0)
    plsc.subcore_barrier()
    pltpu.sync_copy(acc_sh.at[pl.ds(s * STRIPE, STRIPE)],
                    acc_out.at[c, pl.ds(s * STRIPE, STRIPE)])


@functools.cache
def _sc_kernels():
    mesh = plsc.VectorSubcoreMesh(core_axis_name="c", subcore_axis_name="s",
                                  num_cores=2, num_subcores=16)
    deg = pl.kernel(
        _deg_body,
        out_type=jax.ShapeDtypeStruct((2, NP), jnp.float32),
        mesh=mesh,
        scratch_types=[
            pltpu.VMEM((40, 128), jnp.int32),
            pltpu.VMEM((128,), jnp.float32),
            pltpu.VMEM_SHARED((NP,), jnp.float32),
        ],
    )
    agg = pl.kernel(
        _agg_body,
        out_type=jax.ShapeDtypeStruct((2, NP, HALF), jnp.float32),
        mesh=mesh,
        scratch_types=[
            pltpu.VMEM((_NCHUNK, 128), jnp.int32),
            pltpu.VMEM((_NCHUNK, 128), jnp.int32),
            pltpu.VMEM((128, HALF), jnp.float32),
            pltpu.VMEM_SHARED((NP, HALF), jnp.float32),
            pltpu.SemaphoreType.DMA,
        ],
    )
    return deg, agg


def _dinv(d0_ref, d1_ref):
    return lax.rsqrt(d0_ref[...] + d1_ref[...] + 1.0)


def _leaky(z):
    return jnp.where(z >= 0, z, 0.01 * z)


def _mm_body(x_ref, w_ref, d0_ref, d1_ref, g_ref):
    u = x_ref[...] * _dinv(d0_ref, d1_ref)
    h = jnp.dot(u, w_ref[...], preferred_element_type=jnp.float32)
    g_ref[0] = h[:, :HALF]
    g_ref[1] = h[:, HALF:]


def _mid_body(acc_ref, g_ref, d0_ref, d1_ref, b_ref, w_ref, out_ref):
    dinv = _dinv(d0_ref, d1_ref)
    z0 = _leaky(dinv * (acc_ref[0] + g_ref[0]) + b_ref[0:1, :HALF])
    z1 = _leaky(dinv * (acc_ref[1] + g_ref[1]) + b_ref[0:1, HALF:])
    h = (jnp.dot(dinv * z0, w_ref[:HALF, :], preferred_element_type=jnp.float32)
         + jnp.dot(dinv * z1, w_ref[HALF:, :], preferred_element_type=jnp.float32))
    out_ref[0] = h[:, :HALF]
    out_ref[1] = h[:, HALF:]


def _fin_body(acc_ref, g_ref, d0_ref, d1_ref, b_ref, out_ref):
    dinv = _dinv(d0_ref, d1_ref)
    out_ref[:, :HALF] = _leaky(dinv * (acc_ref[0] + g_ref[0]) + b_ref[0:1, :HALF])
    out_ref[:, HALF:] = _leaky(dinv * (acc_ref[1] + g_ref[1]) + b_ref[0:1, HALF:])


_GRID = (NP // BN,)
_SPEC_ROWS = pl.BlockSpec((BN, D), lambda i: (i, 0))
_SPEC_W = pl.BlockSpec((D, D), lambda i: (0, 0))
_SPEC_D = pl.BlockSpec((BN, 1), lambda i: (i, 0))
_SPEC_B = pl.BlockSpec((1, D), lambda i: (0, 0))
_SPEC_HALVES = pl.BlockSpec((2, BN, HALF), lambda i: (0, i, 0))

_mm_call = pl.pallas_call(
    _mm_body,
    grid=_GRID,
    in_specs=[_SPEC_ROWS, _SPEC_W, _SPEC_D, _SPEC_D],
    out_specs=_SPEC_HALVES,
    out_shape=jax.ShapeDtypeStruct((2, NP, HALF), jnp.float32),
)

_mid_call = pl.pallas_call(
    _mid_body,
    grid=_GRID,
    in_specs=[_SPEC_HALVES, _SPEC_HALVES, _SPEC_D, _SPEC_D, _SPEC_B,
              _SPEC_W],
    out_specs=_SPEC_HALVES,
    out_shape=jax.ShapeDtypeStruct((2, NP, HALF), jnp.float32),
)

_fin_call = pl.pallas_call(
    _fin_body,
    grid=_GRID,
    in_specs=[_SPEC_HALVES, _SPEC_HALVES, _SPEC_D, _SPEC_D, _SPEC_B],
    out_specs=_SPEC_ROWS,
    out_shape=jax.ShapeDtypeStruct((NP, D), jnp.float32),
)


def kernel(x, edge_index, W1, b1, W2, b2):
    ei = edge_index.astype(jnp.int32)
    srcp = jnp.concatenate([ei[0], jnp.zeros((EPAD - E,), jnp.int32)])
    dstp = jnp.concatenate([ei[1], jnp.full((EPAD - E,), TRASH, jnp.int32)])
    src2 = jnp.stack([srcp, srcp + NP]).reshape(2, 16, _NCHUNK, 128)
    dst16 = dstp.reshape(16, _NCHUNK, 128)
    dst32 = dstp.reshape(32, 40, 128)
    xpad = jnp.zeros((NP, D), jnp.float32).at[:N].set(x)
    zeros1 = jnp.zeros((NP,), jnp.float32)
    zeros2 = jnp.zeros((NP, HALF), jnp.float32)
    b1r = b1.reshape(1, D)
    b2r = b2.reshape(1, D)

    _deg, _agg = _sc_kernels()
    deg = _deg(dst32, zeros1)
    d0 = deg[0][:, None]
    d1 = deg[1][:, None]
    g1 = _mm_call(xpad, W1, d0, d1)
    acc1 = _agg(g1.reshape(2 * NP, HALF), src2, dst16, zeros2)
    g2 = _mid_call(acc1, g1, d0, d1, b1r, W2)
    acc2 = _agg(g2.reshape(2 * NP, HALF), src2, dst16, zeros2)
    out = _fin_call(acc2, g2, d0, d1, b2r)
    return out[:N]


# speedup vs baseline: 1.9499x; 1.0232x over previous
import functools

import jax
import jax.numpy as jnp
from jax import lax
from jax.experimental import pallas as pl
from jax.experimental.pallas import tpu as pltpu
from jax.experimental.pallas import tpu_sc as plsc

N = 10000
D = 256
E = 160000
NP = 10240
EPAD = 163840
TRASH = 10000
HALF = 128
STRIPE = NP // 16
BN = 1024
_NCHUNK = 80


def _deg_body(dst32, zeros1, deg_out, idx_v, ones_v, deg_sh):
    c = lax.axis_index("c")
    s = lax.axis_index("s")
    w = c * 16 + s
    pltpu.sync_copy(zeros1.at[pl.ds(s * STRIPE, STRIPE)],
                    deg_sh.at[pl.ds(s * STRIPE, STRIPE)])
    pltpu.sync_copy(dst32.at[w], idx_v)
    for k in range(8):
        ones_v[pl.ds(k * 16, 16)] = jnp.full((16,), 1.0, jnp.float32)
    plsc.subcore_barrier()

    def body(j, carry):
        pltpu.sync_copy(ones_v, deg_sh.at[idx_v.at[j]], add=True)
        return carry

    lax.fori_loop(0, 40, body, 0)
    plsc.subcore_barrier()
    pltpu.sync_copy(deg_sh.at[pl.ds(s * STRIPE, STRIPE)],
                    deg_out.at[c, pl.ds(s * STRIPE, STRIPE)])


def _agg_body(g_hbm, src2, dst16, zeros2, acc_out, idxs_v, idxd_v,
              rows_v, acc_sh, sem):
    c = lax.axis_index("c")
    s = lax.axis_index("s")
    pltpu.sync_copy(zeros2.at[pl.ds(s * STRIPE, STRIPE)],
                    acc_sh.at[pl.ds(s * STRIPE, STRIPE)])
    pltpu.sync_copy(src2.at[c, s], idxs_v)
    pltpu.sync_copy(dst16.at[s], idxd_v)
    plsc.subcore_barrier()

    def body(j, carry):
        pltpu.async_copy(g_hbm.at[idxs_v.at[j]], rows_v, sem).wait()
        pltpu.sync_copy(rows_v, acc_sh.at[idxd_v.at[j]], add=True)
        return carry

    lax.fori_loop(0, _NCHUNK, body, 0)
    plsc.subcore_barrier()
    pltpu.sync_copy(acc_sh.at[pl.ds(s * STRIPE, STRIPE)],
                    acc_out.at[c, pl.ds(s * STRIPE, STRIPE)])


@functools.cache
def _sc_kernels():
    mesh = plsc.VectorSubcoreMesh(core_axis_name="c", subcore_axis_name="s",
                                  num_cores=2, num_subcores=16)
    deg = pl.kernel(
        _deg_body,
        out_type=jax.ShapeDtypeStruct((2, NP), jnp.float32),
        mesh=mesh,
        scratch_types=[
            pltpu.VMEM((40, 128), jnp.int32),
            pltpu.VMEM((128,), jnp.float32),
            pltpu.VMEM_SHARED((NP,), jnp.float32),
        ],
    )
    agg = pl.kernel(
        _agg_body,
        out_type=jax.ShapeDtypeStruct((2, NP, HALF), jnp.float32),
        mesh=mesh,
        scratch_types=[
            pltpu.VMEM((_NCHUNK, 128), jnp.int32),
            pltpu.VMEM((_NCHUNK, 128), jnp.int32),
            pltpu.VMEM((128, HALF), jnp.float32),
            pltpu.VMEM_SHARED((NP, HALF), jnp.float32),
            pltpu.SemaphoreType.DMA,
        ],
    )
    return deg, agg


def _dinv(d0_ref, d1_ref):
    return lax.rsqrt(d0_ref[...] + d1_ref[...] + 1.0)


def _leaky(z):
    return jnp.where(z >= 0, z, 0.01 * z)


def _mm_body(x_ref, w_ref, d0_ref, d1_ref, g_ref):
    u = x_ref[...] * _dinv(d0_ref, d1_ref)
    h = jnp.dot(u, w_ref[...], preferred_element_type=jnp.float32)
    g_ref[0] = h[:, :HALF]
    g_ref[1] = h[:, HALF:]


def _mid_body(acc_ref, g_ref, d0_ref, d1_ref, b_ref, w_ref, out_ref):
    dinv = _dinv(d0_ref, d1_ref)
    z0 = _leaky(dinv * (acc_ref[0] + g_ref[0]) + b_ref[0:1, :HALF])
    z1 = _leaky(dinv * (acc_ref[1] + g_ref[1]) + b_ref[0:1, HALF:])
    h = (jnp.dot(dinv * z0, w_ref[:HALF, :], preferred_element_type=jnp.float32)
         + jnp.dot(dinv * z1, w_ref[HALF:, :], preferred_element_type=jnp.float32))
    out_ref[0] = h[:, :HALF]
    out_ref[1] = h[:, HALF:]


def _fin_body(acc_ref, g_ref, d0_ref, d1_ref, b_ref, out_ref):
    dinv = _dinv(d0_ref, d1_ref)
    out_ref[:, :HALF] = _leaky(dinv * (acc_ref[0] + g_ref[0]) + b_ref[0:1, :HALF])
    out_ref[:, HALF:] = _leaky(dinv * (acc_ref[1] + g_ref[1]) + b_ref[0:1, HALF:])


_GRID = (NP // BN,)
_SPEC_ROWS = pl.BlockSpec((BN, D), lambda i: (i, 0))
_SPEC_W = pl.BlockSpec((D, D), lambda i: (0, 0))
_SPEC_D = pl.BlockSpec((BN, 1), lambda i: (i, 0))
_SPEC_B = pl.BlockSpec((1, D), lambda i: (0, 0))
_SPEC_HALVES = pl.BlockSpec((2, BN, HALF), lambda i: (0, i, 0))

_mm_call = pl.pallas_call(
    _mm_body,
    grid=_GRID,
    in_specs=[_SPEC_ROWS, _SPEC_W, _SPEC_D, _SPEC_D],
    out_specs=_SPEC_HALVES,
    out_shape=jax.ShapeDtypeStruct((2, NP, HALF), jnp.float32),
)

_mid_call = pl.pallas_call(
    _mid_body,
    grid=_GRID,
    in_specs=[_SPEC_HALVES, _SPEC_HALVES, _SPEC_D, _SPEC_D, _SPEC_B,
              _SPEC_W],
    out_specs=_SPEC_HALVES,
    out_shape=jax.ShapeDtypeStruct((2, NP, HALF), jnp.float32),
)

_fin_call = pl.pallas_call(
    _fin_body,
    grid=_GRID,
    in_specs=[_SPEC_HALVES, _SPEC_HALVES, _SPEC_D, _SPEC_D, _SPEC_B],
    out_specs=_SPEC_ROWS,
    out_shape=jax.ShapeDtypeStruct((NP, D), jnp.float32),
)


def kernel(x, edge_index, W1, b1, W2, b2):
    ei = edge_index.astype(jnp.int32)
    srcp = jnp.concatenate([ei[0], jnp.zeros((EPAD - E,), jnp.int32)])
    dstp = jnp.concatenate([ei[1], jnp.full((EPAD - E,), TRASH, jnp.int32)])
    src2 = jnp.stack([srcp, srcp + NP]).reshape(2, 16, _NCHUNK, 128)
    dst16 = dstp.reshape(16, _NCHUNK, 128)
    dst32 = dstp.reshape(32, 40, 128)
    xpad = jnp.zeros((NP, D), jnp.float32).at[:N].set(x)
    zeros1 = jnp.zeros((NP,), jnp.float32)
    zeros2 = jnp.zeros((NP, HALF), jnp.float32)
    b1r = b1.reshape(1, D)
    b2r = b2.reshape(1, D)

    _deg, _agg = _sc_kernels()
    deg = _deg(dst32, zeros1)
    d0 = deg[0][:, None]
    d1 = deg[1][:, None]
    g1 = _mm_call(xpad, W1, d0, d1)
    acc1 = _agg(g1.reshape(2 * NP, HALF), src2, dst16, zeros2)
    g2 = _mid_call(acc1, g1, d0, d1, b1r, W2)
    acc2 = _agg(g2.reshape(2 * NP, HALF), src2, dst16, zeros2)
    out = _fin_call(acc2, g2, d0, d1, b2r)
    return out[:N]


# speedup vs baseline: 1.9575x; 1.0039x over previous
import functools

import jax
import jax.numpy as jnp
from jax import lax
from jax.experimental import pallas as pl
from jax.experimental.pallas import tpu as pltpu
from jax.experimental.pallas import tpu_sc as plsc

N = 10000
D = 256
E = 160000
NP = 10240
EPAD = 163840
TRASH = 10000
HALF = 128
STRIPE = NP // 16
BN = 2048
_NCHUNK = 80


def _deg_body(dst32, zeros1, deg_out, idx_v, ones_v, deg_sh):
    c = lax.axis_index("c")
    s = lax.axis_index("s")
    w = c * 16 + s
    pltpu.sync_copy(zeros1.at[pl.ds(s * STRIPE, STRIPE)],
                    deg_sh.at[pl.ds(s * STRIPE, STRIPE)])
    pltpu.sync_copy(dst32.at[w], idx_v)
    for k in range(8):
        ones_v[pl.ds(k * 16, 16)] = jnp.full((16,), 1.0, jnp.float32)
    plsc.subcore_barrier()

    def body(j, carry):
        pltpu.sync_copy(ones_v, deg_sh.at[idx_v.at[j]], add=True)
        return carry

    lax.fori_loop(0, 40, body, 0)
    plsc.subcore_barrier()
    pltpu.sync_copy(deg_sh.at[pl.ds(s * STRIPE, STRIPE)],
                    deg_out.at[c, pl.ds(s * STRIPE, STRIPE)])


def _agg_body(g_hbm, src2, dst16, zeros2, acc_out, idxs_v, idxd_v,
              rows_v, acc_sh, sem):
    c = lax.axis_index("c")
    s = lax.axis_index("s")
    pltpu.sync_copy(zeros2.at[pl.ds(s * STRIPE, STRIPE)],
                    acc_sh.at[pl.ds(s * STRIPE, STRIPE)])
    pltpu.sync_copy(src2.at[c, s], idxs_v)
    pltpu.sync_copy(dst16.at[s], idxd_v)
    plsc.subcore_barrier()

    def body(j, carry):
        pltpu.async_copy(g_hbm.at[idxs_v.at[j]], rows_v, sem).wait()
        pltpu.sync_copy(rows_v, acc_sh.at[idxd_v.at[j]], add=True)
        return carry

    lax.fori_loop(0, _NCHUNK, body, 0)
    plsc.subcore_barrier()
    pltpu.sync_copy(acc_sh.at[pl.ds(s * STRIPE, STRIPE)],
                    acc_out.at[c, pl.ds(s * STRIPE, STRIPE)])


@functools.cache
def _sc_kernels():
    mesh = plsc.VectorSubcoreMesh(core_axis_name="c", subcore_axis_name="s",
                                  num_cores=2, num_subcores=16)
    deg = pl.kernel(
        _deg_body,
        out_type=jax.ShapeDtypeStruct((2, NP), jnp.float32),
        mesh=mesh,
        scratch_types=[
            pltpu.VMEM((40, 128), jnp.int32),
            pltpu.VMEM((128,), jnp.float32),
            pltpu.VMEM_SHARED((NP,), jnp.float32),
        ],
    )
    agg = pl.kernel(
        _agg_body,
        out_type=jax.ShapeDtypeStruct((2, NP, HALF), jnp.float32),
        mesh=mesh,
        scratch_types=[
            pltpu.VMEM((_NCHUNK, 128), jnp.int32),
            pltpu.VMEM((_NCHUNK, 128), jnp.int32),
            pltpu.VMEM((128, HALF), jnp.float32),
            pltpu.VMEM_SHARED((NP, HALF), jnp.float32),
            pltpu.SemaphoreType.DMA,
        ],
    )
    return deg, agg


def _dinv(d0_ref, d1_ref):
    return lax.rsqrt(d0_ref[...] + d1_ref[...] + 1.0)


def _leaky(z):
    return jnp.where(z >= 0, z, 0.01 * z)


def _mm_body(x_ref, w_ref, d0_ref, d1_ref, g_ref):
    u = x_ref[...] * _dinv(d0_ref, d1_ref)
    h = jnp.dot(u, w_ref[...], preferred_element_type=jnp.float32)
    g_ref[0] = h[:, :HALF]
    g_ref[1] = h[:, HALF:]


def _mid_body(acc_ref, g_ref, d0_ref, d1_ref, b_ref, w_ref, out_ref):
    dinv = _dinv(d0_ref, d1_ref)
    z0 = _leaky(dinv * (acc_ref[0] + g_ref[0]) + b_ref[0:1, :HALF])
    z1 = _leaky(dinv * (acc_ref[1] + g_ref[1]) + b_ref[0:1, HALF:])
    h = (jnp.dot(dinv * z0, w_ref[:HALF, :], preferred_element_type=jnp.float32)
         + jnp.dot(dinv * z1, w_ref[HALF:, :], preferred_element_type=jnp.float32))
    out_ref[0] = h[:, :HALF]
    out_ref[1] = h[:, HALF:]


def _fin_body(acc_ref, g_ref, d0_ref, d1_ref, b_ref, out_ref):
    dinv = _dinv(d0_ref, d1_ref)
    out_ref[:, :HALF] = _leaky(dinv * (acc_ref[0] + g_ref[0]) + b_ref[0:1, :HALF])
    out_ref[:, HALF:] = _leaky(dinv * (acc_ref[1] + g_ref[1]) + b_ref[0:1, HALF:])


_GRID = (NP // BN,)
_SPEC_ROWS = pl.BlockSpec((BN, D), lambda i: (i, 0))
_SPEC_W = pl.BlockSpec((D, D), lambda i: (0, 0))
_SPEC_D = pl.BlockSpec((BN, 1), lambda i: (i, 0))
_SPEC_B = pl.BlockSpec((1, D), lambda i: (0, 0))
_SPEC_HALVES = pl.BlockSpec((2, BN, HALF), lambda i: (0, i, 0))

_mm_call = pl.pallas_call(
    _mm_body,
    grid=_GRID,
    in_specs=[_SPEC_ROWS, _SPEC_W, _SPEC_D, _SPEC_D],
    out_specs=_SPEC_HALVES,
    out_shape=jax.ShapeDtypeStruct((2, NP, HALF), jnp.float32),
)

_mid_call = pl.pallas_call(
    _mid_body,
    grid=_GRID,
    in_specs=[_SPEC_HALVES, _SPEC_HALVES, _SPEC_D, _SPEC_D, _SPEC_B,
              _SPEC_W],
    out_specs=_SPEC_HALVES,
    out_shape=jax.ShapeDtypeStruct((2, NP, HALF), jnp.float32),
)

_fin_call = pl.pallas_call(
    _fin_body,
    grid=_GRID,
    in_specs=[_SPEC_HALVES, _SPEC_HALVES, _SPEC_D, _SPEC_D, _SPEC_B],
    out_specs=_SPEC_ROWS,
    out_shape=jax.ShapeDtypeStruct((NP, D), jnp.float32),
)


def kernel(x, edge_index, W1, b1, W2, b2):
    ei = edge_index.astype(jnp.int32)
    srcp = jnp.concatenate([ei[0], jnp.zeros((EPAD - E,), jnp.int32)])
    dstp = jnp.concatenate([ei[1], jnp.full((EPAD - E,), TRASH, jnp.int32)])
    src2 = jnp.stack([srcp, srcp + NP]).reshape(2, 16, _NCHUNK, 128)
    dst16 = dstp.reshape(16, _NCHUNK, 128)
    dst32 = dstp.reshape(32, 40, 128)
    xpad = jnp.zeros((NP, D), jnp.float32).at[:N].set(x)
    zeros1 = jnp.zeros((NP,), jnp.float32)
    zeros2 = jnp.zeros((NP, HALF), jnp.float32)
    b1r = b1.reshape(1, D)
    b2r = b2.reshape(1, D)

    _deg, _agg = _sc_kernels()
    deg = _deg(dst32, zeros1)
    d0 = deg[0][:, None]
    d1 = deg[1][:, None]
    g1 = _mm_call(xpad, W1, d0, d1)
    acc1 = _agg(g1.reshape(2 * NP, HALF), src2, dst16, zeros2)
    g2 = _mid_call(acc1, g1, d0, d1, b1r, W2)
    acc2 = _agg(g2.reshape(2 * NP, HALF), src2, dst16, zeros2)
    out = _fin_call(acc2, g2, d0, d1, b2r)
    return out[:N]


# speedup vs baseline: 1.9592x; 1.0008x over previous
import functools

import jax
import jax.numpy as jnp
from jax import lax
from jax.experimental import pallas as pl
from jax.experimental.pallas import tpu as pltpu
from jax.experimental.pallas import tpu_sc as plsc

N = 10000
D = 256
E = 160000
NP = 10240
EPAD = 163840
TRASH = 10000
HALF = 128
STRIPE = NP // 16
BN = 5120
_NCHUNK = 80


def _deg_body(dst32, zeros1, deg_out, idx_v, ones_v, deg_sh):
    c = lax.axis_index("c")
    s = lax.axis_index("s")
    w = c * 16 + s
    pltpu.sync_copy(zeros1.at[pl.ds(s * STRIPE, STRIPE)],
                    deg_sh.at[pl.ds(s * STRIPE, STRIPE)])
    pltpu.sync_copy(dst32.at[w], idx_v)
    for k in range(8):
        ones_v[pl.ds(k * 16, 16)] = jnp.full((16,), 1.0, jnp.float32)
    plsc.subcore_barrier()

    def body(j, carry):
        pltpu.sync_copy(ones_v, deg_sh.at[idx_v.at[j]], add=True)
        return carry

    lax.fori_loop(0, 40, body, 0)
    plsc.subcore_barrier()
    pltpu.sync_copy(deg_sh.at[pl.ds(s * STRIPE, STRIPE)],
                    deg_out.at[c, pl.ds(s * STRIPE, STRIPE)])


def _agg_body(g_hbm, src2, dst16, zeros2, acc_out, idxs_v, idxd_v,
              rows_v, acc_sh, sem):
    c = lax.axis_index("c")
    s = lax.axis_index("s")
    pltpu.sync_copy(zeros2.at[pl.ds(s * STRIPE, STRIPE)],
                    acc_sh.at[pl.ds(s * STRIPE, STRIPE)])
    pltpu.sync_copy(src2.at[c, s], idxs_v)
    pltpu.sync_copy(dst16.at[s], idxd_v)
    plsc.subcore_barrier()

    def body(j, carry):
        pltpu.async_copy(g_hbm.at[idxs_v.at[j]], rows_v, sem).wait()
        pltpu.sync_copy(rows_v, acc_sh.at[idxd_v.at[j]], add=True)
        return carry

    lax.fori_loop(0, _NCHUNK, body, 0)
    plsc.subcore_barrier()
    pltpu.sync_copy(acc_sh.at[pl.ds(s * STRIPE, STRIPE)],
                    acc_out.at[c, pl.ds(s * STRIPE, STRIPE)])


@functools.cache
def _sc_kernels():
    mesh = plsc.VectorSubcoreMesh(core_axis_name="c", subcore_axis_name="s",
                                  num_cores=2, num_subcores=16)
    deg = pl.kernel(
        _deg_body,
        out_type=jax.ShapeDtypeStruct((2, NP), jnp.float32),
        mesh=mesh,
        scratch_types=[
            pltpu.VMEM((40, 128), jnp.int32),
            pltpu.VMEM((128,), jnp.float32),
            pltpu.VMEM_SHARED((NP,), jnp.float32),
        ],
    )
    agg = pl.kernel(
        _agg_body,
        out_type=jax.ShapeDtypeStruct((2, NP, HALF), jnp.float32),
        mesh=mesh,
        scratch_types=[
            pltpu.VMEM((_NCHUNK, 128), jnp.int32),
            pltpu.VMEM((_NCHUNK, 128), jnp.int32),
            pltpu.VMEM((128, HALF), jnp.float32),
            pltpu.VMEM_SHARED((NP, HALF), jnp.float32),
            pltpu.SemaphoreType.DMA,
        ],
    )
    return deg, agg


def _dinv(d0_ref, d1_ref):
    return lax.rsqrt(d0_ref[...] + d1_ref[...] + 1.0)


def _leaky(z):
    return jnp.where(z >= 0, z, 0.01 * z)


def _mm_body(x_ref, w_ref, d0_ref, d1_ref, g_ref):
    u = x_ref[...] * _dinv(d0_ref, d1_ref)
    h = jnp.dot(u, w_ref[...], preferred_element_type=jnp.float32)
    g_ref[0] = h[:, :HALF]
    g_ref[1] = h[:, HALF:]


def _mid_body(acc_ref, g_ref, d0_ref, d1_ref, b_ref, w_ref, out_ref):
    dinv = _dinv(d0_ref, d1_ref)
    z0 = _leaky(dinv * (acc_ref[0] + g_ref[0]) + b_ref[0:1, :HALF])
    z1 = _leaky(dinv * (acc_ref[1] + g_ref[1]) + b_ref[0:1, HALF:])
    h = (jnp.dot(dinv * z0, w_ref[:HALF, :], preferred_element_type=jnp.float32)
         + jnp.dot(dinv * z1, w_ref[HALF:, :], preferred_element_type=jnp.float32))
    out_ref[0] = h[:, :HALF]
    out_ref[1] = h[:, HALF:]


def _fin_body(acc_ref, g_ref, d0_ref, d1_ref, b_ref, out_ref):
    dinv = _dinv(d0_ref, d1_ref)
    out_ref[:, :HALF] = _leaky(dinv * (acc_ref[0] + g_ref[0]) + b_ref[0:1, :HALF])
    out_ref[:, HALF:] = _leaky(dinv * (acc_ref[1] + g_ref[1]) + b_ref[0:1, HALF:])


_GRID = (NP // BN,)
_SPEC_ROWS = pl.BlockSpec((BN, D), lambda i: (i, 0))
_SPEC_W = pl.BlockSpec((D, D), lambda i: (0, 0))
_SPEC_D = pl.BlockSpec((BN, 1), lambda i: (i, 0))
_SPEC_B = pl.BlockSpec((1, D), lambda i: (0, 0))
_SPEC_HALVES = pl.BlockSpec((2, BN, HALF), lambda i: (0, i, 0))

_mm_call = pl.pallas_call(
    _mm_body,
    grid=_GRID,
    in_specs=[_SPEC_ROWS, _SPEC_W, _SPEC_D, _SPEC_D],
    out_specs=_SPEC_HALVES,
    out_shape=jax.ShapeDtypeStruct((2, NP, HALF), jnp.float32),
)

_mid_call = pl.pallas_call(
    _mid_body,
    grid=_GRID,
    in_specs=[_SPEC_HALVES, _SPEC_HALVES, _SPEC_D, _SPEC_D, _SPEC_B,
              _SPEC_W],
    out_specs=_SPEC_HALVES,
    out_shape=jax.ShapeDtypeStruct((2, NP, HALF), jnp.float32),
)

_fin_call = pl.pallas_call(
    _fin_body,
    grid=_GRID,
    in_specs=[_SPEC_HALVES, _SPEC_HALVES, _SPEC_D, _SPEC_D, _SPEC_B],
    out_specs=_SPEC_ROWS,
    out_shape=jax.ShapeDtypeStruct((NP, D), jnp.float32),
)


def kernel(x, edge_index, W1, b1, W2, b2):
    ei = edge_index.astype(jnp.int32)
    srcp = jnp.concatenate([ei[0], jnp.zeros((EPAD - E,), jnp.int32)])
    dstp = jnp.concatenate([ei[1], jnp.full((EPAD - E,), TRASH, jnp.int32)])
    src2 = jnp.stack([srcp, srcp + NP]).reshape(2, 16, _NCHUNK, 128)
    dst16 = dstp.reshape(16, _NCHUNK, 128)
    dst32 = dstp.reshape(32, 40, 128)
    xpad = jnp.zeros((NP, D), jnp.float32).at[:N].set(x)
    zeros1 = jnp.zeros((NP,), jnp.float32)
    zeros2 = jnp.zeros((NP, HALF), jnp.float32)
    b1r = b1.reshape(1, D)
    b2r = b2.reshape(1, D)

    _deg, _agg = _sc_kernels()
    deg = _deg(dst32, zeros1)
    d0 = deg[0][:, None]
    d1 = deg[1][:, None]
    g1 = _mm_call(xpad, W1, d0, d1)
    acc1 = _agg(g1.reshape(2 * NP, HALF), src2, dst16, zeros2)
    g2 = _mid_call(acc1, g1, d0, d1, b1r, W2)
    acc2 = _agg(g2.reshape(2 * NP, HALF), src2, dst16, zeros2)
    out = _fin_call(acc2, g2, d0, d1, b2r)
    return out[:N]
